# baseline (speedup 1.0000x reference)
import functools

import jax
import jax.numpy as jnp
from jax import lax
from jax.experimental import pallas as pl
from jax.experimental.pallas import tpu as pltpu
from jax.experimental.pallas import tpu_sc as plsc

_EMBED_DIMS = [(33, 17), (2, 1), (8, 4), (3, 2), (3, 2), (3, 2), (3, 2), (3, 2), (20, 10)]
_VOFFS = []
_COFFS = []
_v = 0
_c = 0
for _vv, _dd in _EMBED_DIMS:
    _VOFFS.append(_v)
    _COFFS.append(_c)
    _v += _vv
    _c += _dd
_TOTV = _v
_TOTC = _c
_CPAD = 48
_NT = len(_EMBED_DIMS)

_NC, _NS = 2, 16
_NW = _NC * _NS


def _sc_body(bpw, nb, tpad_hbm, cat_hbm, out_hbm, catv, tflat, accf):
    wid = lax.axis_index("s") * _NC + lax.axis_index("c")
    base = wid * bpw
    pltpu.sync_copy(tpad_hbm, tflat)
    for i in range(_NT):
        pltpu.sync_copy(cat_hbm.at[pl.ds(i * nb + base, bpw)],
                        catv.at[pl.ds(i * bpw, bpw)])

    stiota = lax.iota(jnp.int32, 16) * _CPAD
    zeros16 = jnp.zeros((16,), jnp.float32)

    def body(g, c):
        sbase = g * (16 * _CPAD)
        for i in range(_NT):
            cv = catv[pl.ds(i * bpw + g * 16, 16)]
            fi = cv * _CPAD + (_VOFFS[i] * _CPAD + _COFFS[i])
            for r in range(_EMBED_DIMS[i][1]):
                vals = plsc.load_gather(tflat, [fi + r])
                plsc.store_scatter(accf, [stiota + (sbase + _COFFS[i] + r)], vals)
        for j in range(_TOTC, _CPAD):
            plsc.store_scatter(accf, [stiota + (sbase + j)], zeros16)
        return c

    lax.fori_loop(0, bpw // 16, body, 0)
    pltpu.sync_copy(accf, out_hbm.at[pl.ds(base * _CPAD, bpw * _CPAD)])


def _tc_body(nb, xb_ref, xr_ref, e_ref, w_ref, wc_ref, b_ref, g_ref, be_ref, o_ref):
    xr = xr_ref[...]
    mean = jnp.sum(xr) * (1.0 / nb)
    var = jnp.sum((xr - mean) ** 2) * (1.0 / nb)
    rstd = jax.lax.rsqrt(var + 1e-5)

    xb = xb_ref[...]
    cn = (xb[:, 0:1] - mean) * rstd * g_ref[0, 0] + be_ref[0, 0]

    z = jax.lax.dot_general(
        e_ref[...], w_ref[...], (((1,), (1,)), ((), ())),
        preferred_element_type=jnp.float32, precision=jax.lax.Precision.HIGHEST)
    z = z + cn * wc_ref[...] + b_ref[...]
    z = z - jnp.max(z, axis=1, keepdims=True)
    ez = jnp.exp(z)
    o_ref[...] = ez / jnp.sum(ez, axis=1, keepdims=True)


def kernel(x, emb0, emb1, emb2, emb3, emb4, emb5, emb6, emb7, emb8, W, b, gamma, beta):
    tables = [emb0, emb1, emb2, emb3, emb4, emb5, emb6, emb7, emb8]
    B = x.shape[0]
    d_out = W.shape[0]
    bpw = B // _NW

    tpad = jnp.zeros((_TOTV, _CPAD), jnp.float32)
    for i, t in enumerate(tables):
        v, d = t.shape
        tpad = tpad.at[_VOFFS[i]:_VOFFS[i] + v, _COFFS[i]:_COFFS[i] + d].set(t)

    cat_t = x[:, 1:].astype(jnp.int32).T

    mesh = plsc.VectorSubcoreMesh(core_axis_name="c", subcore_axis_name="s")
    ef = pl.kernel(
        functools.partial(_sc_body, bpw, B),
        out_type=jax.ShapeDtypeStruct((B * _CPAD,), jnp.float32),
        mesh=mesh,
        scratch_types=[
            pltpu.VMEM((_NT * bpw,), jnp.int32),
            pltpu.VMEM((_TOTV * _CPAD,), jnp.float32),
            pltpu.VMEM((bpw * _CPAD,), jnp.float32),
        ],
        compiler_params=pltpu.CompilerParams(needs_layout_passes=False),
    )(tpad.reshape(-1), cat_t.reshape(-1))
    e = ef.reshape(B, _CPAD)

    w_pad = jnp.zeros((d_out, _CPAD), jnp.float32).at[:, :_TOTC].set(W[:, :_TOTC])
    wc = W[:, _TOTC].reshape(1, d_out)
    xr = x[:, 0].reshape(128, B // 128)
    b2 = b.reshape(1, d_out)
    g2 = gamma.reshape(1, 1)
    be2 = beta.reshape(1, 1)

    bb = 1024
    out = pl.pallas_call(
        functools.partial(_tc_body, float(B)),
        grid=(B // bb,),
        in_specs=[
            pl.BlockSpec((bb, x.shape[1]), lambda i: (i, 0)),
            pl.BlockSpec(xr.shape, lambda i: (0, 0)),
            pl.BlockSpec((bb, _CPAD), lambda i: (i, 0)),
            pl.BlockSpec(w_pad.shape, lambda i: (0, 0)),
            pl.BlockSpec(wc.shape, lambda i: (0, 0)),
            pl.BlockSpec(b2.shape, lambda i: (0, 0)),
            pl.BlockSpec(g2.shape, lambda i: (0, 0)),
            pl.BlockSpec(be2.shape, lambda i: (0, 0)),
        ],
        out_specs=pl.BlockSpec((bb, d_out), lambda i: (i, 0)),
        out_shape=jax.ShapeDtypeStruct((B, d_out), jnp.float32),
    )(x, xr, e, w_pad, wc, b2, g2, be2)
    return out


# speedup vs baseline: 1.0859x; 1.0859x over previous
import functools

import jax
import jax.numpy as jnp
from jax import lax
from jax.experimental import pallas as pl
from jax.experimental.pallas import tpu as pltpu
from jax.experimental.pallas import tpu_sc as plsc

_EMBED_DIMS = [(33, 17), (2, 1), (8, 4), (3, 2), (3, 2), (3, 2), (3, 2), (3, 2), (20, 10)]
_VOFFS = []
_COFFS = []
_v = 0
_c = 0
for _vv, _dd in _EMBED_DIMS:
    _VOFFS.append(_v)
    _COFFS.append(_c)
    _v += _vv
    _c += _dd
_TOTV = _v
_TOTC = _c
_CPAD = 48
_NT = len(_EMBED_DIMS)

_NC, _NS = 2, 16
_NW = _NC * _NS


def _sc_body(bpw, nb, tpad_hbm, cat_hbm, out_hbm, catv, tflat, accT):
    wid = lax.axis_index("s") * _NC + lax.axis_index("c")
    base = wid * bpw
    pltpu.sync_copy(tpad_hbm, tflat)
    for i in range(_NT):
        pltpu.sync_copy(cat_hbm.at[pl.ds(i * nb + base, bpw)],
                        catv.at[pl.ds(i * bpw, bpw)])

    zeros16 = jnp.zeros((16,), jnp.float32)
    for j in range(_TOTC, _CPAD):
        for g in range(bpw // 16):
            accT[pl.ds(j * bpw + g * 16, 16)] = zeros16

    @plsc.parallel_loop(0, bpw // 16, unroll=4)
    def _loop(g):
        for i in range(_NT):
            cv = catv[pl.ds(i * bpw + g * 16, 16)]
            fi = cv * _CPAD + (_VOFFS[i] * _CPAD + _COFFS[i])
            for r in range(_EMBED_DIMS[i][1]):
                vals = plsc.load_gather(tflat, [fi + r])
                accT[pl.ds((_COFFS[i] + r) * bpw + g * 16, 16)] = vals

    for j in range(_CPAD):
        pltpu.sync_copy(accT.at[pl.ds(j * bpw, bpw)],
                        out_hbm.at[pl.ds(j * nb + base, bpw)])


def _tc_body(nb, xb_ref, xr_ref, et_ref, w_ref, wc_ref, b_ref, g_ref, be_ref, o_ref):
    xr = xr_ref[...]
    mean = jnp.sum(xr) * (1.0 / nb)
    var = jnp.sum((xr - mean) ** 2) * (1.0 / nb)
    rstd = jax.lax.rsqrt(var + 1e-5)

    xb = xb_ref[...]
    cn = (xb[:, 0:1] - mean) * rstd * g_ref[0, 0] + be_ref[0, 0]

    z = jax.lax.dot_general(
        et_ref[...], w_ref[...], (((0,), (1,)), ((), ())),
        preferred_element_type=jnp.float32, precision=jax.lax.Precision.HIGHEST)
    z = z + cn * wc_ref[...] + b_ref[...]
    z = z - jnp.max(z, axis=1, keepdims=True)
    ez = jnp.exp(z)
    o_ref[...] = ez / jnp.sum(ez, axis=1, keepdims=True)


def kernel(x, emb0, emb1, emb2, emb3, emb4, emb5, emb6, emb7, emb8, W, b, gamma, beta):
    tables = [emb0, emb1, emb2, emb3, emb4, emb5, emb6, emb7, emb8]
    B = x.shape[0]
    d_out = W.shape[0]
    bpw = B // _NW

    tpad = jnp.zeros((_TOTV, _CPAD), jnp.float32)
    for i, t in enumerate(tables):
        v, d = t.shape
        tpad = tpad.at[_VOFFS[i]:_VOFFS[i] + v, _COFFS[i]:_COFFS[i] + d].set(t)

    cat_t = x[:, 1:].astype(jnp.int32).T

    mesh = plsc.VectorSubcoreMesh(core_axis_name="c", subcore_axis_name="s")
    ef = pl.kernel(
        functools.partial(_sc_body, bpw, B),
        out_type=jax.ShapeDtypeStruct((_CPAD * B,), jnp.float32),
        mesh=mesh,
        scratch_types=[
            pltpu.VMEM((_NT * bpw,), jnp.int32),
            pltpu.VMEM((_TOTV * _CPAD,), jnp.float32),
            pltpu.VMEM((bpw * _CPAD,), jnp.float32),
        ],
        compiler_params=pltpu.CompilerParams(needs_layout_passes=False),
    )(tpad.reshape(-1), cat_t.reshape(-1))
    et = ef.reshape(_CPAD, B)

    w_pad = jnp.zeros((d_out, _CPAD), jnp.float32).at[:, :_TOTC].set(W[:, :_TOTC])
    wc = W[:, _TOTC].reshape(1, d_out)
    xr = x[:, 0].reshape(128, B // 128)
    b2 = b.reshape(1, d_out)
    g2 = gamma.reshape(1, 1)
    be2 = beta.reshape(1, 1)

    bb = 1024
    out = pl.pallas_call(
        functools.partial(_tc_body, float(B)),
        grid=(B // bb,),
        in_specs=[
            pl.BlockSpec((bb, x.shape[1]), lambda i: (i, 0)),
            pl.BlockSpec(xr.shape, lambda i: (0, 0)),
            pl.BlockSpec((_CPAD, bb), lambda i: (0, i)),
            pl.BlockSpec(w_pad.shape, lambda i: (0, 0)),
            pl.BlockSpec(wc.shape, lambda i: (0, 0)),
            pl.BlockSpec(b2.shape, lambda i: (0, 0)),
            pl.BlockSpec(g2.shape, lambda i: (0, 0)),
            pl.BlockSpec(be2.shape, lambda i: (0, 0)),
        ],
        out_specs=pl.BlockSpec((bb, d_out), lambda i: (i, 0)),
        out_shape=jax.ShapeDtypeStruct((B, d_out), jnp.float32),
        compiler_params=pltpu.CompilerParams(fuse_transposed_lhs_in_matmul=True),
    )(x, xr, et, w_pad, wc, b2, g2, be2)
    return out


# speedup vs baseline: 1.1876x; 1.0937x over previous
import functools

import jax
import jax.numpy as jnp
from jax import lax
from jax.experimental import pallas as pl
from jax.experimental.pallas import tpu as pltpu
from jax.experimental.pallas import tpu_sc as plsc

_EMBED_DIMS = [(33, 17), (2, 1), (8, 4), (3, 2), (3, 2), (3, 2), (3, 2), (3, 2), (20, 10)]
_VOFFS = []
_COFFS = []
_v = 0
_c = 0
for _vv, _dd in _EMBED_DIMS:
    _VOFFS.append(_v)
    _COFFS.append(_c)
    _v += _vv
    _c += _dd
_TOTV = _v
_TOTC = _c
_CPAD = 48
_NT = len(_EMBED_DIMS)

_NC, _NS = 2, 16
_NW = _NC * _NS


def _sc_body(bpw, nb, tpad_hbm, cat_hbm, out_hbm, catv, tflat, accT, sem_in, sem_out):
    wid = lax.axis_index("s") * _NC + lax.axis_index("c")
    base = wid * bpw
    loads = [pltpu.async_copy(tpad_hbm, tflat, sem_in)]
    for i in range(_NT):
        loads.append(pltpu.async_copy(cat_hbm.at[pl.ds(i * nb + base, bpw)],
                                      catv.at[pl.ds(i * bpw, bpw)], sem_in))
    for cp in loads:
        cp.wait()

    zeros16 = jnp.zeros((16,), jnp.float32)
    for j in range(_TOTC, _CPAD):
        for g in range(bpw // 16):
            accT[pl.ds(j * bpw + g * 16, 16)] = zeros16

    @plsc.parallel_loop(0, bpw // 16, unroll=4)
    def _loop(g):
        for i in range(_NT):
            cv = catv[pl.ds(i * bpw + g * 16, 16)]
            fi = cv * _CPAD + (_VOFFS[i] * _CPAD + _COFFS[i])
            for r in range(_EMBED_DIMS[i][1]):
                vals = plsc.load_gather(tflat, [fi + r])
                accT[pl.ds((_COFFS[i] + r) * bpw + g * 16, 16)] = vals

    stores = [pltpu.async_copy(accT.at[pl.ds(j * bpw, bpw)],
                               out_hbm.at[pl.ds(j * nb + base, bpw)], sem_out)
              for j in range(_CPAD)]
    for cp in stores:
        cp.wait()


def _tc_body(nb, xb_ref, xr_ref, et_ref, w_ref, wc_ref, b_ref, g_ref, be_ref, o_ref):
    xr = xr_ref[...]
    mean = jnp.sum(xr) * (1.0 / nb)
    var = jnp.sum((xr - mean) ** 2) * (1.0 / nb)
    rstd = jax.lax.rsqrt(var + 1e-5)

    xb = xb_ref[...]
    cn = (xb[:, 0:1] - mean) * rstd * g_ref[0, 0] + be_ref[0, 0]

    z = jax.lax.dot_general(
        et_ref[...], w_ref[...], (((0,), (1,)), ((), ())),
        preferred_element_type=jnp.float32, precision=jax.lax.Precision.HIGHEST)
    z = z + cn * wc_ref[...] + b_ref[...]
    z = z - jnp.max(z, axis=1, keepdims=True)
    ez = jnp.exp(z)
    o_ref[...] = ez / jnp.sum(ez, axis=1, keepdims=True)


def kernel(x, emb0, emb1, emb2, emb3, emb4, emb5, emb6, emb7, emb8, W, b, gamma, beta):
    tables = [emb0, emb1, emb2, emb3, emb4, emb5, emb6, emb7, emb8]
    B = x.shape[0]
    d_out = W.shape[0]
    bpw = B // _NW

    tpad = jnp.zeros((_TOTV, _CPAD), jnp.float32)
    for i, t in enumerate(tables):
        v, d = t.shape
        tpad = tpad.at[_VOFFS[i]:_VOFFS[i] + v, _COFFS[i]:_COFFS[i] + d].set(t)

    cat_t = x[:, 1:].astype(jnp.int32).T

    mesh = plsc.VectorSubcoreMesh(core_axis_name="c", subcore_axis_name="s")
    ef = pl.kernel(
        functools.partial(_sc_body, bpw, B),
        out_type=jax.ShapeDtypeStruct((_CPAD * B,), jnp.float32),
        mesh=mesh,
        scratch_types=[
            pltpu.VMEM((_NT * bpw,), jnp.int32),
            pltpu.VMEM((_TOTV * _CPAD,), jnp.float32),
            pltpu.VMEM((bpw * _CPAD,), jnp.float32),
            pltpu.SemaphoreType.DMA,
            pltpu.SemaphoreType.DMA,
        ],
        compiler_params=pltpu.CompilerParams(needs_layout_passes=False),
    )(tpad.reshape(-1), cat_t.reshape(-1))
    et = ef.reshape(_CPAD, B)

    w_pad = jnp.zeros((d_out, _CPAD), jnp.float32).at[:, :_TOTC].set(W[:, :_TOTC])
    wc = W[:, _TOTC].reshape(1, d_out)
    xr = x[:, 0].reshape(128, B // 128)
    b2 = b.reshape(1, d_out)
    g2 = gamma.reshape(1, 1)
    be2 = beta.reshape(1, 1)

    bb = 1024
    out = pl.pallas_call(
        functools.partial(_tc_body, float(B)),
        grid=(B // bb,),
        in_specs=[
            pl.BlockSpec((bb, x.shape[1]), lambda i: (i, 0)),
            pl.BlockSpec(xr.shape, lambda i: (0, 0)),
            pl.BlockSpec((_CPAD, bb), lambda i: (0, i)),
            pl.BlockSpec(w_pad.shape, lambda i: (0, 0)),
            pl.BlockSpec(wc.shape, lambda i: (0, 0)),
            pl.BlockSpec(b2.shape, lambda i: (0, 0)),
            pl.BlockSpec(g2.shape, lambda i: (0, 0)),
            pl.BlockSpec(be2.shape, lambda i: (0, 0)),
        ],
        out_specs=pl.BlockSpec((bb, d_out), lambda i: (i, 0)),
        out_shape=jax.ShapeDtypeStruct((B, d_out), jnp.float32),
        compiler_params=pltpu.CompilerParams(fuse_transposed_lhs_in_matmul=True),
    )(x, xr, et, w_pad, wc, b2, g2, be2)
    return out
